# baseline (speedup 1.0000x reference)
import jax
import jax.numpy as jnp
from jax import lax
from jax.experimental import pallas as pl
from jax.experimental.pallas import tpu as pltpu
from jax.experimental.pallas import tpu_sc as plsc

N, D, R, K, P = 10000, 256, 4, 8, 4096
L = 16
NW = 32
ROWS_W = 2 * P // NW
RK = R * K
BKT_W = ROWS_W * R
GB = 128
NGROUP = BKT_W // GB
NBUF = 2
IDX_W = BKT_W * K
ESEC = 256
NESEC = IDX_W // ESEC


def _sc_nbr_body(pairs_hbm, nbr_hbm, out_hbm, pair_v, nbr_v,
                 sem_n, sem_o):
    wid = lax.axis_index("s") * 2 + lax.axis_index("c")
    base = wid * ROWS_W
    pltpu.sync_copy(pairs_hbm.at[pl.ds(base, ROWS_W)], pair_v)
    pltpu.make_async_copy(nbr_hbm.at[pair_v], nbr_v, sem_n).start()
    pltpu.make_async_copy(nbr_hbm.at[pair_v], nbr_v, sem_n).wait()
    pltpu.make_async_copy(nbr_v, out_hbm.at[pl.ds(base, ROWS_W)], sem_o).start()
    pltpu.make_async_copy(nbr_v, out_hbm.at[pl.ds(base, ROWS_W)], sem_o).wait()


@jax.jit
def _sc_nbr_gather(pair_nodes, nbr2d):
    mesh = plsc.VectorSubcoreMesh(core_axis_name="c", subcore_axis_name="s")
    return pl.kernel(
        _sc_nbr_body,
        out_type=jax.ShapeDtypeStruct((2 * P, RK), jnp.int32),
        mesh=mesh,
        compiler_params=pltpu.CompilerParams(use_tc_tiling_on_sc=False),
        scratch_types=[
            pltpu.VMEM((ROWS_W,), jnp.int32),
            pltpu.VMEM((ROWS_W, RK), jnp.int32),
            pltpu.SemaphoreType.DMA,
            pltpu.SemaphoreType.DMA,
        ],
    )(pair_nodes, nbr2d)


def _sc_body(fidx_hbm, nbr1d_hbm, emb_hbm, out_hbm,
             fidx_v, nidx_v, table_sp, accs, gsems, osems, isem, sem_tab):
    cid = lax.axis_index("c")
    sid = lax.axis_index("s")
    wid = sid * 2 + cid
    bkt_base = wid * BKT_W

    tab_rows = N // (NW // 2)
    tab_cp = pltpu.make_async_copy(
        emb_hbm.at[pl.ds(sid * tab_rows, tab_rows)],
        table_sp.at[pl.ds(sid * tab_rows, tab_rows)], sem_tab)
    tab_cp.start()

    pltpu.sync_copy(fidx_hbm.at[pl.ds(wid * IDX_W, IDX_W)], fidx_v)

    def esec_cp(s):
        return pltpu.make_async_copy(
            nbr1d_hbm.at[fidx_v.at[pl.ds(s * ESEC, ESEC)]],
            nidx_v.at[pl.ds(s * ESEC, ESEC)], isem)

    for s in range(NESEC):
        esec_cp(s).start()

    tab_cp.wait()
    plsc.subcore_barrier()
    spg = K * GB // ESEC

    def add_cp(g, b, k):
        idx = nidx_v.at[pl.ds((g * K + k) * GB, GB)]
        return pltpu.make_async_copy(table_sp.at[idx], accs[b], gsems[b])

    def out_cp(g, b):
        return pltpu.make_async_copy(
            accs[b], out_hbm.at[pl.ds(bkt_base + g * GB, GB)], osems[b])

    zv = jnp.zeros((2 * L,), jnp.bfloat16)

    def zero(b):
        def zrow(rw, _):
            for cc in range(D // (2 * L)):
                accs[b][rw, pl.ds(cc * 2 * L, 2 * L)] = zv
            return _
        lax.fori_loop(0, GB, zrow, None)

    def fire_adds(g, b):
        for k in range(K):
            idx = nidx_v.at[pl.ds((g * K + k) * GB, GB)]
            pltpu.async_copy(table_sp.at[idx], accs[b], gsems[b], add=True)

    for b in range(NBUF):
        for s in range(b * spg, (b + 1) * spg):
            esec_cp(s).wait()
        zero(b)
        fire_adds(b, b)

    def step(i, _):
        for b in range(NBUF):
            g = i * NBUF + b
            for k in range(K):
                add_cp(g, b, k).wait()
            out_cp(g, b).start()

            @pl.when(g + NBUF < NGROUP)
            def _():
                for ss in range(spg):
                    esec_cp((g + NBUF) * spg + ss).wait()
                out_cp(g, b).wait()
                zero(b)
                fire_adds(g + NBUF, b)
        return _

    lax.fori_loop(0, NGROUP // NBUF, step, None)
    for b in range(NBUF):
        out_cp(NGROUP - NBUF + b, b).wait()


@jax.jit
def _sc_gather_sum(fidx, nbr1d, node_embeds):
    mesh = plsc.VectorSubcoreMesh(core_axis_name="c", subcore_axis_name="s")
    return pl.kernel(
        _sc_body,
        out_type=jax.ShapeDtypeStruct((2 * P * R, D), jnp.bfloat16),
        mesh=mesh,
        compiler_params=pltpu.CompilerParams(use_tc_tiling_on_sc=False),
        scratch_types=[
            pltpu.VMEM((IDX_W,), jnp.int32),
            pltpu.VMEM((IDX_W,), jnp.int32),
            pltpu.VMEM_SHARED((N, D), jnp.bfloat16),
            [pltpu.VMEM((GB, D), jnp.bfloat16) for _ in range(NBUF)],
            [pltpu.SemaphoreType.DMA for _ in range(NBUF)],
            [pltpu.SemaphoreType.DMA for _ in range(NBUF)],
            pltpu.SemaphoreType.DMA,
            pltpu.SemaphoreType.DMA,
        ],
    )(fidx, nbr1d, node_embeds)


def _tc_body(g_ref, w_ref, b_ref, o_ref):
    x = g_ref[:, 0, :] + g_ref[:, 1, :]
    acc = jnp.dot(x, w_ref[...], preferred_element_type=jnp.float32)
    o_ref[...] = jax.nn.sigmoid(acc + b_ref[...])


def _tc_matmul(g3, w_cat, bias):
    blk = 512
    return pl.pallas_call(
        _tc_body,
        grid=(P // blk,),
        in_specs=[
            pl.BlockSpec((blk, 2, R * D), lambda i: (i, 0, 0)),
            pl.BlockSpec((R * D, D), lambda i: (0, 0)),
            pl.BlockSpec((1, D), lambda i: (0, 0)),
        ],
        out_specs=pl.BlockSpec((blk, D), lambda i: (i, 0)),
        out_shape=jax.ShapeDtypeStruct((P, D), jnp.float32),
    )(g3, w_cat, bias)


def _build_fidx():
    import numpy as np
    w = np.arange(NW)[:, None, None, None]
    b = (np.arange(NGROUP)[None, :, None, None] * GB
         + np.arange(GB)[None, None, None, :])
    k = np.arange(K)[None, None, :, None]
    t = w * ROWS_W + b // R
    r = b % R
    return np.broadcast_to(t * RK + r * K + k,
                           (NW, NGROUP, K, GB)).reshape(-1).astype(np.int32)


_FIDX = _build_fidx()


def kernel(node_pairs, node_embeds, node_types, neighbor_data, W_beta_w, W_beta_b):
    del node_types
    pair_nodes = node_pairs.reshape(-1).astype(jnp.int32)
    nbr2d = neighbor_data.reshape(N, RK).astype(jnp.int32)
    nidx = _sc_nbr_gather(pair_nodes, nbr2d).reshape(-1)
    fidx = jnp.asarray(_FIDX)
    g = _sc_gather_sum(fidx, nidx, node_embeds.astype(jnp.bfloat16))
    w_cat = (jnp.transpose(W_beta_w, (0, 2, 1)).reshape(R * D, D)
             * (1.0 / (2 * K * R))).astype(jnp.bfloat16)
    bias = jnp.mean(W_beta_b, axis=0, keepdims=True)
    return _tc_matmul(g.reshape(P, 2, R * D), w_cat, bias)


# speedup vs baseline: 1.0252x; 1.0252x over previous
import jax
import jax.numpy as jnp
from jax import lax
from jax.experimental import pallas as pl
from jax.experimental.pallas import tpu as pltpu
from jax.experimental.pallas import tpu_sc as plsc

N, D, R, K, P = 10000, 256, 4, 8, 4096
L = 16
NW = 32
ROWS_W = 2 * P // NW
RK = R * K
BKT_W = ROWS_W * R
GB = 128
NGROUP = BKT_W // GB
NBUF = 2
IDX_W = BKT_W * K
ESEC = 256
NESEC = IDX_W // ESEC


def _sc_nbr_body(pairs_hbm, nbr_hbm, out_hbm, pair_v, nbr_v,
                 sem_n, sem_o):
    wid = lax.axis_index("s") * 2 + lax.axis_index("c")
    base = wid * ROWS_W
    pltpu.sync_copy(pairs_hbm.at[pl.ds(base, ROWS_W)], pair_v)
    pltpu.make_async_copy(nbr_hbm.at[pair_v], nbr_v, sem_n).start()
    pltpu.make_async_copy(nbr_hbm.at[pair_v], nbr_v, sem_n).wait()
    pltpu.make_async_copy(nbr_v, out_hbm.at[pl.ds(base, ROWS_W)], sem_o).start()
    pltpu.make_async_copy(nbr_v, out_hbm.at[pl.ds(base, ROWS_W)], sem_o).wait()


@jax.jit
def _sc_nbr_gather(pair_nodes, nbr2d):
    mesh = plsc.VectorSubcoreMesh(core_axis_name="c", subcore_axis_name="s")
    return pl.kernel(
        _sc_nbr_body,
        out_type=jax.ShapeDtypeStruct((2 * P, RK), jnp.int32),
        mesh=mesh,
        compiler_params=pltpu.CompilerParams(use_tc_tiling_on_sc=False),
        scratch_types=[
            pltpu.VMEM((ROWS_W,), jnp.int32),
            pltpu.VMEM((ROWS_W, RK), jnp.int32),
            pltpu.SemaphoreType.DMA,
            pltpu.SemaphoreType.DMA,
        ],
    )(pair_nodes, nbr2d)


def _sc_body(fidx_hbm, nbr1d_hbm, emb_hbm, out_hbm,
             fidx_v, nidx_v, table_sp, accs, gsems, osems, isem, sem_tab):
    cid = lax.axis_index("c")
    sid = lax.axis_index("s")
    wid = sid * 2 + cid
    bkt_base = wid * BKT_W

    tab_rows = N // (NW // 2)
    tab_cp = pltpu.make_async_copy(
        emb_hbm.at[pl.ds(sid * tab_rows, tab_rows)],
        table_sp.at[pl.ds(sid * tab_rows, tab_rows)], sem_tab)
    tab_cp.start()

    pltpu.sync_copy(fidx_hbm.at[pl.ds(wid * IDX_W, IDX_W)], fidx_v)

    def esec_cp(s):
        return pltpu.make_async_copy(
            nbr1d_hbm.at[fidx_v.at[pl.ds(s * ESEC, ESEC)]],
            nidx_v.at[pl.ds(s * ESEC, ESEC)], isem)

    for s in range(NESEC):
        esec_cp(s).start()
    for s in range(NESEC):
        esec_cp(s).wait()

    tab_cp.wait()
    plsc.subcore_barrier()

    def add_cp(g, b, k):
        idx = nidx_v.at[pl.ds((g * K + k) * GB, GB)]
        return pltpu.make_async_copy(table_sp.at[idx], accs[b], gsems[b])

    def out_cp(g, b):
        return pltpu.make_async_copy(
            accs[b], out_hbm.at[pl.ds(bkt_base + g * GB, GB)], osems[b])

    zv = jnp.zeros((2 * L,), jnp.bfloat16)

    def zero(b):
        def zrow(rw, _):
            for cc in range(D // (2 * L)):
                accs[b][rw, pl.ds(cc * 2 * L, 2 * L)] = zv
            return _
        lax.fori_loop(0, GB, zrow, None)

    def fire_adds(g, b):
        for k in range(K):
            idx = nidx_v.at[pl.ds((g * K + k) * GB, GB)]
            pltpu.async_copy(table_sp.at[idx], accs[b], gsems[b], add=True)

    for b in range(NBUF):
        zero(b)
        fire_adds(b, b)

    def step(i, _):
        for b in range(NBUF):
            g = i * NBUF + b
            for k in range(K):
                add_cp(g, b, k).wait()
            out_cp(g, b).start()

            @pl.when(g + NBUF < NGROUP)
            def _():
                out_cp(g, b).wait()
                zero(b)
                fire_adds(g + NBUF, b)
        return _

    lax.fori_loop(0, NGROUP // NBUF, step, None)
    for b in range(NBUF):
        out_cp(NGROUP - NBUF + b, b).wait()


@jax.jit
def _sc_gather_sum(fidx, nbr1d, node_embeds):
    mesh = plsc.VectorSubcoreMesh(core_axis_name="c", subcore_axis_name="s")
    return pl.kernel(
        _sc_body,
        out_type=jax.ShapeDtypeStruct((2 * P * R, D), jnp.bfloat16),
        mesh=mesh,
        compiler_params=pltpu.CompilerParams(use_tc_tiling_on_sc=False),
        scratch_types=[
            pltpu.VMEM((IDX_W,), jnp.int32),
            pltpu.VMEM((IDX_W,), jnp.int32),
            pltpu.VMEM_SHARED((N, D), jnp.bfloat16),
            [pltpu.VMEM((GB, D), jnp.bfloat16) for _ in range(NBUF)],
            [pltpu.SemaphoreType.DMA for _ in range(NBUF)],
            [pltpu.SemaphoreType.DMA for _ in range(NBUF)],
            pltpu.SemaphoreType.DMA,
            pltpu.SemaphoreType.DMA,
        ],
    )(fidx, nbr1d, node_embeds)


def _tc_body(g_ref, w_ref, b_ref, o_ref):
    x = g_ref[:, 0, :] + g_ref[:, 1, :]
    acc = jnp.dot(x, w_ref[...], preferred_element_type=jnp.float32)
    o_ref[...] = jax.nn.sigmoid(acc + b_ref[...])


def _tc_matmul(g3, w_cat, bias):
    blk = 512
    return pl.pallas_call(
        _tc_body,
        grid=(P // blk,),
        in_specs=[
            pl.BlockSpec((blk, 2, R * D), lambda i: (i, 0, 0)),
            pl.BlockSpec((R * D, D), lambda i: (0, 0)),
            pl.BlockSpec((1, D), lambda i: (0, 0)),
        ],
        out_specs=pl.BlockSpec((blk, D), lambda i: (i, 0)),
        out_shape=jax.ShapeDtypeStruct((P, D), jnp.float32),
    )(g3, w_cat, bias)


def _build_fidx():
    import numpy as np
    w = np.arange(NW)[:, None, None, None]
    b = (np.arange(NGROUP)[None, :, None, None] * GB
         + np.arange(GB)[None, None, None, :])
    k = np.arange(K)[None, None, :, None]
    t = w * ROWS_W + b // R
    r = b % R
    return np.broadcast_to(t * RK + r * K + k,
                           (NW, NGROUP, K, GB)).reshape(-1).astype(np.int32)


_FIDX = _build_fidx()


def kernel(node_pairs, node_embeds, node_types, neighbor_data, W_beta_w, W_beta_b):
    del node_types
    pair_nodes = node_pairs.reshape(-1).astype(jnp.int32)
    nbr2d = neighbor_data.reshape(N, RK).astype(jnp.int32)
    nidx = _sc_nbr_gather(pair_nodes, nbr2d).reshape(-1)
    fidx = jnp.asarray(_FIDX)
    g = _sc_gather_sum(fidx, nidx, node_embeds.astype(jnp.bfloat16))
    w_cat = (jnp.transpose(W_beta_w, (0, 2, 1)).reshape(R * D, D)
             * (1.0 / (2 * K * R))).astype(jnp.bfloat16)
    bias = jnp.mean(W_beta_b, axis=0, keepdims=True)
    return _tc_matmul(g.reshape(P, 2, R * D), w_cat, bias)
